# baseline (speedup 1.0000x reference)
import functools

import jax
import jax.numpy as jnp
from jax import lax
from jax.experimental import pallas as pl
from jax.experimental.pallas import tpu as pltpu
from jax.experimental.pallas import tpu_sc as plsc

N_ROWS = 100000
B = 16384
D = 64
W = 128
L = 16

_info = plsc.get_sparse_core_info()
NC, NS = _info.num_cores, _info.num_subcores
NW = NC * NS
BPW = B // NW
NROUND = 2
RB = BPW // NROUND
NGROUP = RB // L

_NBT = 32768
_HBT = _NBT // 2
_LOGH = _HBT.bit_length() - 1
_NBLK = (N_ROWS + _NBT - 1) // _NBT
_PACKED_ROWS = _NBLK * _HBT


def _pack_body(xt_ref, out_ref):
    y = xt_ref[...].T
    out_ref[...] = jnp.concatenate([y[:_HBT], y[_HBT:]], axis=1)


def _pack_pairs(table_t):
    return pl.pallas_call(
        _pack_body,
        grid=(_NBLK,),
        in_specs=[pl.BlockSpec((D, _NBT), lambda i: (0, i))],
        out_specs=pl.BlockSpec((_HBT, W), lambda i: (i, 0)),
        out_shape=jax.ShapeDtypeStruct((_PACKED_ROWS, W), jnp.float32),
    )(table_t)


def _packed_id(i):
    return ((i >> (_LOGH + 1)) << _LOGH) | (i & (_HBT - 1))


def _half_base(i):
    return ((i >> _LOGH) & 1) << 6


def _ugather_body(users_hbm, uemb_hbm, uout_hbm,
                  uidx_v, suidx_v, urows_v, ucmp_v, sem):
    wid = lax.axis_index("s") * NC + lax.axis_index("c")
    base = wid * BPW
    pltpu.sync_copy(users_hbm.at[pl.ds(base, BPW)], uidx_v)

    iota = lax.broadcasted_iota(jnp.int32, (L,), 0)

    def shift(i, _):
        s = pl.ds(i * L, L)
        suidx_v[s] = _packed_id(uidx_v[s])
        return 0

    lax.fori_loop(0, BPW // L, shift, 0)

    copies = []
    for j in range(BPW // 128):
        copies.append(pltpu.async_copy(uemb_hbm.at[suidx_v.at[pl.ds(j * 128, 128)]],
                                       urows_v.at[pl.ds(j * 128, 128)], sem))
    for c in copies:
        c.wait()

    def group(g, _):
        rowv = g * L + iota
        ubase = _half_base(uidx_v[pl.ds(g * L, L)])
        crow = rowv >> 1
        cbase = (rowv & 1) << 6
        for d in range(D):
            colv = iota ^ d
            val = plsc.load_gather(urows_v, [rowv, ubase | colv])
            plsc.store_scatter(ucmp_v, [crow, cbase | colv], val)
        return 0

    lax.fori_loop(0, BPW // L, group, 0)
    pltpu.sync_copy(ucmp_v, uout_hbm.at[pl.ds(wid * (BPW // 2), BPW // 2)])


def _bpr_body(pos_hbm, neg_hbm, users_hbm, iemb_hbm, ucmp_hbm,
              rui_hbm, ruj_hbm, loss_hbm,
              pidx_v, nidx_v, spidx_v, snidx_v,
              prows_v, nrows_v, u_v,
              rui_v, ruj_v, loss_v, sem0, sem1):
    sem = (sem0, sem1)
    wid = lax.axis_index("s") * NC + lax.axis_index("c")
    base = wid * BPW
    pltpu.sync_copy(pos_hbm.at[pl.ds(base, BPW)], pidx_v)
    pltpu.sync_copy(neg_hbm.at[pl.ds(base, BPW)], nidx_v)
    pltpu.sync_copy(ucmp_hbm.at[pl.ds(wid * (BPW // 2), BPW // 2)], u_v)

    iota = lax.broadcasted_iota(jnp.int32, (L,), 0)
    zero = jnp.zeros((L,), jnp.float32)

    def shift(i, _):
        s = pl.ds(i * L, L)
        spidx_v[s] = _packed_id(pidx_v[s])
        snidx_v[s] = _packed_id(nidx_v[s])
        return 0

    lax.fori_loop(0, BPW // L, shift, 0)

    NCH = BPW // 128

    def issue(k):
        src = pl.ds(k * 128, 128)
        dst = pl.ds((k % 2) * 128, 128)
        s = sem[k % 2]
        return (pltpu.async_copy(iemb_hbm.at[spidx_v.at[src]],
                                 prows_v.at[dst], s),
                pltpu.async_copy(iemb_hbm.at[snidx_v.at[src]],
                                 nrows_v.at[dst], s))

    inflight = {0: issue(0), 1: issue(1)}

    def chunk_compute(k, carry):
        l1, l2, l3 = carry

        def group(g, carry2):
            l1, l2, l3 = carry2
            rowv = (k % 2) * 128 + g * L + iota
            out = pl.ds(k * 128 + g * L, L)
            pbase = _half_base(pidx_v[out])
            nbase = _half_base(nidx_v[out])
            urowv = (k * 128 + g * L + iota) >> 1
            ubase = ((g * L + iota) & 1) << 6
            rui_a = zero
            rui_b = zero
            ruj_a = zero
            ruj_b = zero
            for d in range(D):
                colv = iota ^ d
                iu = plsc.load_gather(u_v, [urowv, ubase | colv])
                ip = plsc.load_gather(prows_v, [rowv, pbase | colv])
                iv = plsc.load_gather(nrows_v, [rowv, nbase | colv])
                if d % 2 == 0:
                    rui_a = rui_a + iu * ip
                    ruj_a = ruj_a + iu * iv
                else:
                    rui_b = rui_b + iu * ip
                    ruj_b = ruj_b + iu * iv
                l1 = l1 + iu * iu
                l2 = l2 + ip * ip
                l3 = l3 + iv * iv
            rui_v[out] = rui_a + rui_b
            ruj_v[out] = ruj_a + ruj_b
            return (l1, l2, l3)

        return lax.fori_loop(0, 128 // L, group, (l1, l2, l3))

    carry = (zero, zero, zero)
    for k in range(NCH):
        for c in inflight.pop(k):
            c.wait()
        carry = chunk_compute(k, carry)
        if k + 2 < NCH:
            inflight[k + 2] = issue(k + 2)
    l1, l2, l3 = carry
    loss_v[...] = l1 + l2 + l3

    pltpu.sync_copy(rui_v, rui_hbm.at[pl.ds(base, BPW)])
    pltpu.sync_copy(ruj_v, ruj_hbm.at[pl.ds(base, BPW)])
    pltpu.sync_copy(loss_v, loss_hbm.at[wid])


@jax.jit
def _bpr_sc(users, pos_items, neg_items, user_emb, item_emb):
    mesh = plsc.VectorSubcoreMesh(core_axis_name="c", subcore_axis_name="s")
    cparams = pltpu.CompilerParams(needs_layout_passes=False,
                                   use_tc_tiling_on_sc=True)
    k1 = functools.partial(
        pl.kernel,
        mesh=mesh,
        compiler_params=cparams,
        out_type=[jax.ShapeDtypeStruct((B // 2, W), jnp.float32)],
        scratch_types=[
            pltpu.VMEM((BPW,), jnp.int32),
            pltpu.VMEM((BPW,), jnp.int32),
            pltpu.VMEM((BPW, W), jnp.float32),
            pltpu.VMEM((BPW // 2, W), jnp.float32),
            pltpu.SemaphoreType.DMA,
        ],
    )(_ugather_body)
    k2 = functools.partial(
        pl.kernel,
        mesh=mesh,
        compiler_params=cparams,
        out_type=[
            jax.ShapeDtypeStruct((B,), jnp.float32),
            jax.ShapeDtypeStruct((B,), jnp.float32),
            jax.ShapeDtypeStruct((NW, L), jnp.float32),
        ],
        scratch_types=[
            pltpu.VMEM((BPW,), jnp.int32),
            pltpu.VMEM((BPW,), jnp.int32),
            pltpu.VMEM((BPW,), jnp.int32),
            pltpu.VMEM((BPW,), jnp.int32),
            pltpu.VMEM((RB, W), jnp.float32),
            pltpu.VMEM((RB, W), jnp.float32),
            pltpu.VMEM((BPW // 2, W), jnp.float32),
            pltpu.VMEM((BPW,), jnp.float32),
            pltpu.VMEM((BPW,), jnp.float32),
            pltpu.VMEM((L,), jnp.float32),
            pltpu.SemaphoreType.DMA,
            pltpu.SemaphoreType.DMA,
        ],
    )(_bpr_body)
    upk = _pack_pairs(user_emb.T)
    ipk = _pack_pairs(item_emb.T)
    users_i = users.astype(jnp.int32)
    (ucmp,) = k1(users_i, upk)
    rui, ruj, loss_parts = k2(pos_items.astype(jnp.int32),
                              neg_items.astype(jnp.int32), users_i, ipk, ucmp)
    return (rui.reshape(B, 1), ruj.reshape(B, 1), jnp.sum(loss_parts))


def kernel(users, pos_items, neg_items, user_emb, item_emb):
    return _bpr_sc(users, pos_items, neg_items, user_emb, item_emb)


# speedup vs baseline: 1.0863x; 1.0863x over previous
import functools

import jax
import jax.numpy as jnp
from jax import lax
from jax.experimental import pallas as pl
from jax.experimental.pallas import tpu as pltpu
from jax.experimental.pallas import tpu_sc as plsc

N_ROWS = 100000
B = 16384
D = 64
W = 128
L = 16

_info = plsc.get_sparse_core_info()
NC, NS = _info.num_cores, _info.num_subcores
NW = NC * NS
BPW = B // NW
NROUND = 2
RB = BPW // NROUND
NGROUP = RB // L

_NBT = 16384
_HBT = _NBT // 2
_LOGH = _HBT.bit_length() - 1
_NBLK = (N_ROWS + _NBT - 1) // _NBT
_PACKED_ROWS = _NBLK * _HBT


def _pack_body(xt_ref, out_ref):
    y = xt_ref[...].T
    out_ref[...] = jnp.concatenate([y[:_HBT], y[_HBT:]], axis=1)


def _pack_pairs(table_t):
    return pl.pallas_call(
        _pack_body,
        grid=(_NBLK,),
        in_specs=[pl.BlockSpec((D, _NBT), lambda i: (0, i))],
        out_specs=pl.BlockSpec((_HBT, W), lambda i: (i, 0)),
        out_shape=jax.ShapeDtypeStruct((_PACKED_ROWS, W), jnp.float32),
    )(table_t)


def _packed_id(i):
    return ((i >> (_LOGH + 1)) << _LOGH) | (i & (_HBT - 1))


def _half_base(i):
    return ((i >> _LOGH) & 1) << 6


def _ugather_body(users_hbm, uemb_hbm, uout_hbm,
                  uidx_v, suidx_v, urows_v, ucmp_v, sem):
    wid = lax.axis_index("s") * NC + lax.axis_index("c")
    base = wid * BPW
    pltpu.sync_copy(users_hbm.at[pl.ds(base, BPW)], uidx_v)

    iota = lax.broadcasted_iota(jnp.int32, (L,), 0)

    def shift(i, _):
        s = pl.ds(i * L, L)
        suidx_v[s] = _packed_id(uidx_v[s])
        return 0

    lax.fori_loop(0, BPW // L, shift, 0)

    copies = []
    for j in range(BPW // 128):
        copies.append(pltpu.async_copy(uemb_hbm.at[suidx_v.at[pl.ds(j * 128, 128)]],
                                       urows_v.at[pl.ds(j * 128, 128)], sem))
    for c in copies:
        c.wait()

    def group(g, _):
        rowv = g * L + iota
        ubase = _half_base(uidx_v[pl.ds(g * L, L)])
        crow = rowv >> 1
        cbase = (rowv & 1) << 6
        for d in range(D):
            colv = iota ^ d
            val = plsc.load_gather(urows_v, [rowv, ubase | colv])
            plsc.store_scatter(ucmp_v, [crow, cbase | colv], val)
        return 0

    lax.fori_loop(0, BPW // L, group, 0)
    pltpu.sync_copy(ucmp_v, uout_hbm.at[pl.ds(wid * (BPW // 2), BPW // 2)])


def _bpr_body(pos_hbm, neg_hbm, users_hbm, iemb_hbm, ucmp_hbm,
              rui_hbm, ruj_hbm, loss_hbm,
              pidx_v, nidx_v, spidx_v, snidx_v,
              prows_v, nrows_v, u_v,
              rui_v, ruj_v, loss_v, sem0, sem1):
    sem = (sem0, sem1)
    wid = lax.axis_index("s") * NC + lax.axis_index("c")
    base = wid * BPW
    pltpu.sync_copy(pos_hbm.at[pl.ds(base, BPW)], pidx_v)
    pltpu.sync_copy(neg_hbm.at[pl.ds(base, BPW)], nidx_v)
    pltpu.sync_copy(ucmp_hbm.at[pl.ds(wid * (BPW // 2), BPW // 2)], u_v)

    iota = lax.broadcasted_iota(jnp.int32, (L,), 0)
    zero = jnp.zeros((L,), jnp.float32)

    def shift(i, _):
        s = pl.ds(i * L, L)
        spidx_v[s] = _packed_id(pidx_v[s])
        snidx_v[s] = _packed_id(nidx_v[s])
        return 0

    lax.fori_loop(0, BPW // L, shift, 0)

    NCH = BPW // 128

    def issue(k):
        src = pl.ds(k * 128, 128)
        dst = pl.ds((k % 2) * 128, 128)
        s = sem[k % 2]
        return (pltpu.async_copy(iemb_hbm.at[spidx_v.at[src]],
                                 prows_v.at[dst], s),
                pltpu.async_copy(iemb_hbm.at[snidx_v.at[src]],
                                 nrows_v.at[dst], s))

    inflight = {0: issue(0), 1: issue(1)}

    def chunk_compute(k, carry):
        l1, l2, l3 = carry

        def group(g, carry2):
            l1, l2, l3 = carry2
            rowv = (k % 2) * 128 + g * L + iota
            out = pl.ds(k * 128 + g * L, L)
            pbase = _half_base(pidx_v[out])
            nbase = _half_base(nidx_v[out])
            urowv = (k * 128 + g * L + iota) >> 1
            ubase = ((g * L + iota) & 1) << 6
            rui_a = zero
            rui_b = zero
            ruj_a = zero
            ruj_b = zero
            for d in range(D):
                colv = iota ^ d
                iu = plsc.load_gather(u_v, [urowv, ubase | colv])
                ip = plsc.load_gather(prows_v, [rowv, pbase | colv])
                iv = plsc.load_gather(nrows_v, [rowv, nbase | colv])
                if d % 2 == 0:
                    rui_a = rui_a + iu * ip
                    ruj_a = ruj_a + iu * iv
                else:
                    rui_b = rui_b + iu * ip
                    ruj_b = ruj_b + iu * iv
                l1 = l1 + iu * iu
                l2 = l2 + ip * ip
                l3 = l3 + iv * iv
            rui_v[out] = rui_a + rui_b
            ruj_v[out] = ruj_a + ruj_b
            return (l1, l2, l3)

        return lax.fori_loop(0, 128 // L, group, (l1, l2, l3))

    carry = (zero, zero, zero)
    for k in range(NCH):
        for c in inflight.pop(k):
            c.wait()
        carry = chunk_compute(k, carry)
        if k + 2 < NCH:
            inflight[k + 2] = issue(k + 2)
    l1, l2, l3 = carry
    loss_v[...] = l1 + l2 + l3

    pltpu.sync_copy(rui_v, rui_hbm.at[pl.ds(base, BPW)])
    pltpu.sync_copy(ruj_v, ruj_hbm.at[pl.ds(base, BPW)])
    pltpu.sync_copy(loss_v, loss_hbm.at[wid])


@jax.jit
def _bpr_sc(users, pos_items, neg_items, user_emb, item_emb):
    mesh = plsc.VectorSubcoreMesh(core_axis_name="c", subcore_axis_name="s")
    cparams = pltpu.CompilerParams(needs_layout_passes=False,
                                   use_tc_tiling_on_sc=True)
    k1 = functools.partial(
        pl.kernel,
        mesh=mesh,
        compiler_params=cparams,
        out_type=[jax.ShapeDtypeStruct((B // 2, W), jnp.float32)],
        scratch_types=[
            pltpu.VMEM((BPW,), jnp.int32),
            pltpu.VMEM((BPW,), jnp.int32),
            pltpu.VMEM((BPW, W), jnp.float32),
            pltpu.VMEM((BPW // 2, W), jnp.float32),
            pltpu.SemaphoreType.DMA,
        ],
    )(_ugather_body)
    k2 = functools.partial(
        pl.kernel,
        mesh=mesh,
        compiler_params=cparams,
        out_type=[
            jax.ShapeDtypeStruct((B,), jnp.float32),
            jax.ShapeDtypeStruct((B,), jnp.float32),
            jax.ShapeDtypeStruct((NW, L), jnp.float32),
        ],
        scratch_types=[
            pltpu.VMEM((BPW,), jnp.int32),
            pltpu.VMEM((BPW,), jnp.int32),
            pltpu.VMEM((BPW,), jnp.int32),
            pltpu.VMEM((BPW,), jnp.int32),
            pltpu.VMEM((RB, W), jnp.float32),
            pltpu.VMEM((RB, W), jnp.float32),
            pltpu.VMEM((BPW // 2, W), jnp.float32),
            pltpu.VMEM((BPW,), jnp.float32),
            pltpu.VMEM((BPW,), jnp.float32),
            pltpu.VMEM((L,), jnp.float32),
            pltpu.SemaphoreType.DMA,
            pltpu.SemaphoreType.DMA,
        ],
    )(_bpr_body)
    upk = _pack_pairs(user_emb.T)
    ipk = _pack_pairs(item_emb.T)
    users_i = users.astype(jnp.int32)
    (ucmp,) = k1(users_i, upk)
    rui, ruj, loss_parts = k2(pos_items.astype(jnp.int32),
                              neg_items.astype(jnp.int32), users_i, ipk, ucmp)
    return (rui.reshape(B, 1), ruj.reshape(B, 1), jnp.sum(loss_parts))


def kernel(users, pos_items, neg_items, user_emb, item_emb):
    return _bpr_sc(users, pos_items, neg_items, user_emb, item_emb)
